# baseline (speedup 1.0000x reference)
import functools

import numpy as np
import jax
import jax.numpy as jnp
from jax import lax
from jax.experimental import pallas as pl
from jax.experimental.pallas import tpu as pltpu
from jax.experimental.pallas import tpu_sc as plsc

_NF = 26
_D = 16
_B = 16384
_ROWS = _B * _NF
_FIELD = 100000
_ROWS_TBL = _NF * _FIELD
_ZDIM = _NF * _D


def _pair_row(i, j):
    return i * (_NF - 1) - i * (i - 1) // 2 + (j - i - 1)


_PR = []
for _d in range(1, _NF):
    for _i in range(_NF - _d):
        _PR.append(_pair_row(_i, _i + _d))
_GIDX = np.repeat(np.asarray(_PR, np.int32), _D)
_PDIM = _GIDX.shape[0]


_NC = 2
_NS = 16

_NCOL = _ROWS_TBL // 128
_TAIL = _ROWS_TBL - _NCOL * 128
_CPW = 640
_BLK = 8


@functools.cache
def _make_relayout():
    mesh = plsc.VectorSubcoreMesh(core_axis_name="c", subcore_axis_name="s")

    @functools.partial(
        pl.kernel,
        mesh=mesh,
        out_type=jax.ShapeDtypeStruct((_ROWS_TBL // 8, 128), jnp.float32),
        scratch_types=[
            pltpu.VMEM((2, _D, 128 * _BLK), jnp.float32),
            pltpu.VMEM((2, 16 * _BLK, 128), jnp.float32),
            pltpu.SemaphoreType.DMA,
            pltpu.SemaphoreType.DMA,
            pltpu.SemaphoreType.DMA,
            pltpu.SemaphoreType.DMA,
            pltpu.VMEM((_D, _TAIL), jnp.float32),
            pltpu.VMEM((_TAIL // 8, 128), jnp.float32),
        ],
        compiler_params=pltpu.CompilerParams(
            use_tc_tiling_on_sc=True, needs_layout_passes=False),
    )
    def relayout_k(tt_hbm, out_hbm, tin_v, tout_v, is0, is1, os0, os1,
                   ti_v, to_v):
        wid = lax.axis_index("s") * _NC + lax.axis_index("c")
        lanes = lax.iota(jnp.int32, 16)
        c_lo = wid * _CPW
        c_hi = jnp.minimum(c_lo + _CPW, _NCOL)
        nblk = (c_hi - c_lo) // _BLK
        pairs = nblk // 2
        isem = (is0, is1)
        osem = (os0, os1)

        def start_in(slot, b):
            c0 = c_lo + b * _BLK
            pltpu.async_copy(tt_hbm.at[:, pl.ds(c0 * 128, 128 * _BLK)],
                             tin_v.at[slot], isem[slot])

        def wait_in(slot):
            pltpu.make_async_copy(tt_hbm.at[:, pl.ds(0, 128 * _BLK)],
                                  tin_v.at[slot], isem[slot]).wait()

        def start_out(slot, b):
            c0 = c_lo + b * _BLK
            pltpu.async_copy(tout_v.at[slot],
                             out_hbm.at[pl.ds(c0 * 16, 16 * _BLK)],
                             osem[slot])

        def wait_out(slot):
            pltpu.make_async_copy(tout_v.at[slot],
                                  out_hbm.at[pl.ds(0, 16 * _BLK)],
                                  osem[slot]).wait()

        def transpose_blk(slot):
            def body(t, _):
                j = t >> 3
                lr0 = (t & 7) * 16
                row_i = j * 16 + lr0 // 8 + (lanes >> 3)
                col_b = (lanes & 7) * _D
                src0 = j * 128 + lr0
                for k in range(_D):
                    vk = tin_v[slot, k, pl.ds(src0, 16)]
                    plsc.store_scatter(tout_v.at[slot], [row_i, col_b + k], vk)
                return _
            lax.fori_loop(0, 8 * _BLK, body, 0, unroll=4)

        start_in(0, 0)

        def pair_loop(p, carry):
            b0 = 2 * p
            wait_in(0)
            start_in(1, b0 + 1)

            @pl.when(p > 0)
            def _():
                wait_out(0)
            transpose_blk(0)
            start_out(0, b0)
            wait_in(1)

            @pl.when(b0 + 2 < nblk)
            def _():
                start_in(0, b0 + 2)

            @pl.when(p > 0)
            def _():
                wait_out(1)
            transpose_blk(1)
            start_out(1, b0 + 1)
            return carry

        lax.fori_loop(0, pairs, pair_loop, 0)

        @pl.when(nblk % 2 == 1)
        def _():
            wait_in(0)
            wait_out(0)
            transpose_blk(0)
            start_out(0, nblk - 1)
        wait_out(0)
        wait_out(1)

        @pl.when(wid == _NC * _NS - 1)
        def _():
            pltpu.sync_copy(tt_hbm.at[:, pl.ds(_NCOL * 128, _TAIL)], ti_v)
            for t in range(_TAIL // 16):
                lr0 = t * 16
                row_i = lr0 // 8 + (lanes >> 3)
                col_b = (lanes & 7) * _D
                for k in range(_D):
                    vk = ti_v[k, pl.ds(lr0, 16)]
                    plsc.store_scatter(to_v, [row_i, col_b + k], vk)
            pltpu.sync_copy(to_v, out_hbm.at[pl.ds(_NCOL * 16, _TAIL // 8)])

    return relayout_k


@functools.cache
def _make_gather(nrows=_ROWS):
    nw = _NC * _NS
    rows_per_w = nrows // nw
    ch = 416
    nchunk = rows_per_w // ch
    assert rows_per_w % ch == 0 and nchunk % 2 == 0
    mesh = plsc.VectorSubcoreMesh(core_axis_name="c", subcore_axis_name="s")

    @functools.partial(
        pl.kernel,
        mesh=mesh,
        out_type=jax.ShapeDtypeStruct((nrows, _D), jnp.float32),
        scratch_types=[
            pltpu.VMEM((2, ch), jnp.int32),
            pltpu.VMEM((2, ch), jnp.int32),
            pltpu.VMEM((2, ch, 128), jnp.float32),
            pltpu.VMEM((2, ch, _D), jnp.float32),
            pltpu.SemaphoreType.DMA,
            pltpu.SemaphoreType.DMA,
            pltpu.SemaphoreType.DMA,
            pltpu.SemaphoreType.DMA,
            pltpu.SemaphoreType.DMA,
            pltpu.SemaphoreType.DMA,
        ],
        compiler_params=pltpu.CompilerParams(
            use_tc_tiling_on_sc=False, needs_layout_passes=False),
    )
    def gather_k(idx_hbm, table_hbm, out_hbm, idx_v, grp_v, rows_v, pk_v,
                 is0, is1, gs0, gs1, os0, os1):
        wid = lax.axis_index("s") * _NC + lax.axis_index("c")
        lanes = lax.iota(jnp.int32, 16)
        isem = (is0, is1)
        gsem = (gs0, gs1)
        osem = (os0, os1)

        def start_idx(slot, c):
            base = wid * rows_per_w + c * ch
            pltpu.async_copy(idx_hbm.at[pl.ds(base, ch)], idx_v.at[slot],
                             isem[slot])

        def wait_idx(slot):
            pltpu.make_async_copy(idx_hbm.at[pl.ds(0, ch)], idx_v.at[slot],
                                  isem[slot]).wait()

        def prep(slot):
            wait_idx(slot)

            def to_groups(i, carry):
                v = idx_v[slot, pl.ds(i * 16, 16)]
                grp_v[slot, pl.ds(i * 16, 16)] = v >> 3
                return carry

            lax.fori_loop(0, ch // 16, to_groups, 0, unroll=4)
            pltpu.async_copy(table_hbm.at[grp_v.at[slot]], rows_v.at[slot],
                             gsem[slot])

        def fin(slot, c):
            pltpu.make_async_copy(table_hbm.at[grp_v.at[slot]],
                                  rows_v.at[slot], gsem[slot]).wait()

            @pl.when(c >= 2)
            def _():
                pltpu.make_async_copy(pk_v.at[slot],
                                      out_hbm.at[pl.ds(0, ch)],
                                      osem[slot]).wait()

            def extract(i, carry):
                i0 = i * 16
                rows16 = i0 + lanes
                off = (idx_v[slot, pl.ds(i0, 16)] & 7) * _D
                for k in range(_D):
                    vk = plsc.load_gather(rows_v.at[slot], [rows16, off + k])
                    plsc.store_scatter(pk_v.at[slot], [rows16, lanes * 0 + k],
                                       vk)
                return carry

            lax.fori_loop(0, ch // 16, extract, 0, unroll=2)
            base = wid * rows_per_w + c * ch
            pltpu.async_copy(pk_v.at[slot], out_hbm.at[pl.ds(base, ch)],
                             osem[slot])

            @pl.when(c + 2 < nchunk)
            def _():
                start_idx(slot, c + 2)

        start_idx(0, 0)
        prep(0)
        start_idx(1, 1)

        def pair_loop(p, carry):
            c0 = 2 * p

            @pl.when(c0 + 1 < nchunk)
            def _():
                prep(1)
            fin(0, c0)

            @pl.when(c0 + 2 < nchunk)
            def _():
                prep(0)

            @pl.when(c0 + 1 < nchunk)
            def _():
                fin(1, c0 + 1)
            return carry

        lax.fori_loop(0, nchunk // 2, pair_loop, 0)
        pltpu.make_async_copy(pk_v.at[0], out_hbm.at[pl.ds(0, ch)],
                              osem[0]).wait()
        pltpu.make_async_copy(pk_v.at[1], out_hbm.at[pl.ds(0, ch)],
                              osem[1]).wait()

    return gather_k


def _mlp_body(emb_ref, bias_ref, w1z_ref, u_ref, b1_ref, w2_ref, b2_ref,
              w3t_ref, b3_ref, out_ref):
    e = emb_ref[...]
    zb = (e + bias_ref[...]).astype(jnp.bfloat16)
    acc = jnp.dot(zb, w1z_ref[...], preferred_element_type=jnp.float32)
    eb = e.astype(jnp.bfloat16)
    r = 0
    for d in range(1, _NF):
        w = (_NF - d) * _D
        prod = eb[:, :w] * eb[:, d * _D:]
        acc = acc + jnp.dot(prod, u_ref[r:r + w, :],
                            preferred_element_type=jnp.float32)
        r += w
    h1 = jnp.maximum(acc + b1_ref[...], 0.0).astype(jnp.bfloat16)
    h2 = jnp.maximum(
        jnp.dot(h1, w2_ref[...], preferred_element_type=jnp.float32)
        + b2_ref[...], 0.0)
    logit = jnp.sum(h2 * w3t_ref[...], axis=1, keepdims=True) + b3_ref[...]
    out_ref[...] = jax.nn.sigmoid(logit)


def _mlp(emb2, bias2, w1z, u, b1r, w2b, b2r, w3t, b3r):
    nb = emb2.shape[0]
    tb = 512
    return pl.pallas_call(
        _mlp_body,
        grid=(nb // tb,),
        in_specs=[
            pl.BlockSpec((tb, _ZDIM), lambda i: (i, 0)),
            pl.BlockSpec((1, _ZDIM), lambda i: (0, 0)),
            pl.BlockSpec((_ZDIM, 128), lambda i: (0, 0)),
            pl.BlockSpec((_PDIM, 128), lambda i: (0, 0)),
            pl.BlockSpec((1, 128), lambda i: (0, 0)),
            pl.BlockSpec((128, 64), lambda i: (0, 0)),
            pl.BlockSpec((1, 64), lambda i: (0, 0)),
            pl.BlockSpec((1, 64), lambda i: (0, 0)),
            pl.BlockSpec((1, 1), lambda i: (0, 0)),
        ],
        out_specs=pl.BlockSpec((tb, 1), lambda i: (i, 0)),
        out_shape=jax.ShapeDtypeStruct((nb, 1), jnp.float32),
    )(emb2, bias2, w1z, u, b1r, w2b, b2r, w3t, b3r)


def kernel(x, W_emb, bias, W1, b1, W2, b2, W3, b3):
    offs = jnp.arange(_NF, dtype=jnp.int32) * _FIELD
    idx = (x + offs[None, :]).reshape(-1)

    table_rm = _make_relayout()(W_emb.T)

    w1z = W1[:_ZDIM].astype(jnp.bfloat16)
    u = W1[_ZDIM:][jnp.asarray(_GIDX)].astype(jnp.bfloat16)
    bias2 = bias.reshape(1, _ZDIM)
    b1r = b1.reshape(1, -1)
    b2r = b2.reshape(1, -1)
    w3t = W3.reshape(1, -1)
    b3r = b3.reshape(1, 1)
    w2b = W2.astype(jnp.bfloat16)

    half = _ROWS // 2
    gath = _make_gather(half)
    outs = []
    for h in range(2):
        emb = gath(lax.dynamic_slice_in_dim(idx, h * half, half), table_rm)
        emb2 = emb.reshape(_B // 2, _ZDIM)
        outs.append(_mlp(emb2, bias2, w1z, u, b1r, w2b, b2r, w3t, b3r))
    return jnp.concatenate(outs, axis=0)


# speedup vs baseline: 1.0121x; 1.0121x over previous
import functools

import numpy as np
import jax
import jax.numpy as jnp
from jax import lax
from jax.experimental import pallas as pl
from jax.experimental.pallas import tpu as pltpu
from jax.experimental.pallas import tpu_sc as plsc

_NF = 26
_D = 16
_B = 16384
_ROWS = _B * _NF
_FIELD = 100000
_ROWS_TBL = _NF * _FIELD
_ZDIM = _NF * _D


def _pair_row(i, j):
    return i * (_NF - 1) - i * (i - 1) // 2 + (j - i - 1)


_PR = []
for _d in range(1, _NF):
    for _i in range(_NF - _d):
        _PR.append(_pair_row(_i, _i + _d))
_GIDX = np.repeat(np.asarray(_PR, np.int32), _D)
_PDIM = _GIDX.shape[0]


_NC = 2
_NS = 16

_NCOL = _ROWS_TBL // 128
_TAIL = _ROWS_TBL - _NCOL * 128
_CPW = 640
_BLK = 8


@functools.cache
def _make_relayout():
    mesh = plsc.VectorSubcoreMesh(core_axis_name="c", subcore_axis_name="s")

    @functools.partial(
        pl.kernel,
        mesh=mesh,
        out_type=jax.ShapeDtypeStruct((_ROWS_TBL // 8, 128), jnp.float32),
        scratch_types=[
            pltpu.VMEM((2, _D, 128 * _BLK), jnp.float32),
            pltpu.VMEM((2, 16 * _BLK, 128), jnp.float32),
            pltpu.SemaphoreType.DMA,
            pltpu.SemaphoreType.DMA,
            pltpu.SemaphoreType.DMA,
            pltpu.SemaphoreType.DMA,
            pltpu.VMEM((_D, _TAIL), jnp.float32),
            pltpu.VMEM((_TAIL // 8, 128), jnp.float32),
        ],
        compiler_params=pltpu.CompilerParams(
            use_tc_tiling_on_sc=True, needs_layout_passes=False),
    )
    def relayout_k(tt_hbm, out_hbm, tin_v, tout_v, is0, is1, os0, os1,
                   ti_v, to_v):
        wid = lax.axis_index("s") * _NC + lax.axis_index("c")
        lanes = lax.iota(jnp.int32, 16)
        c_lo = wid * _CPW
        c_hi = jnp.minimum(c_lo + _CPW, _NCOL)
        nblk = (c_hi - c_lo) // _BLK
        pairs = nblk // 2
        isem = (is0, is1)
        osem = (os0, os1)

        def start_in(slot, b):
            c0 = c_lo + b * _BLK
            pltpu.async_copy(tt_hbm.at[:, pl.ds(c0 * 128, 128 * _BLK)],
                             tin_v.at[slot], isem[slot])

        def wait_in(slot):
            pltpu.make_async_copy(tt_hbm.at[:, pl.ds(0, 128 * _BLK)],
                                  tin_v.at[slot], isem[slot]).wait()

        def start_out(slot, b):
            c0 = c_lo + b * _BLK
            pltpu.async_copy(tout_v.at[slot],
                             out_hbm.at[pl.ds(c0 * 16, 16 * _BLK)],
                             osem[slot])

        def wait_out(slot):
            pltpu.make_async_copy(tout_v.at[slot],
                                  out_hbm.at[pl.ds(0, 16 * _BLK)],
                                  osem[slot]).wait()

        def transpose_blk(slot):
            def body(t, _):
                j = t >> 3
                lr0 = (t & 7) * 16
                row_i = j * 16 + lr0 // 8 + (lanes >> 3)
                col_b = (lanes & 7) * _D
                src0 = j * 128 + lr0
                for k in range(_D):
                    vk = tin_v[slot, k, pl.ds(src0, 16)]
                    plsc.store_scatter(tout_v.at[slot], [row_i, col_b + k], vk)
                return _
            lax.fori_loop(0, 8 * _BLK, body, 0, unroll=4)

        start_in(0, 0)

        def pair_loop(p, carry):
            b0 = 2 * p
            wait_in(0)
            start_in(1, b0 + 1)

            @pl.when(p > 0)
            def _():
                wait_out(0)
            transpose_blk(0)
            start_out(0, b0)
            wait_in(1)

            @pl.when(b0 + 2 < nblk)
            def _():
                start_in(0, b0 + 2)

            @pl.when(p > 0)
            def _():
                wait_out(1)
            transpose_blk(1)
            start_out(1, b0 + 1)
            return carry

        lax.fori_loop(0, pairs, pair_loop, 0)

        @pl.when(nblk % 2 == 1)
        def _():
            wait_in(0)
            wait_out(0)
            transpose_blk(0)
            start_out(0, nblk - 1)
        wait_out(0)
        wait_out(1)

        @pl.when(wid == _NC * _NS - 1)
        def _():
            pltpu.sync_copy(tt_hbm.at[:, pl.ds(_NCOL * 128, _TAIL)], ti_v)
            for t in range(_TAIL // 16):
                lr0 = t * 16
                row_i = lr0 // 8 + (lanes >> 3)
                col_b = (lanes & 7) * _D
                for k in range(_D):
                    vk = ti_v[k, pl.ds(lr0, 16)]
                    plsc.store_scatter(to_v, [row_i, col_b + k], vk)
            pltpu.sync_copy(to_v, out_hbm.at[pl.ds(_NCOL * 16, _TAIL // 8)])

    return relayout_k


@functools.cache
def _make_gather(nrows=_ROWS):
    nw = _NC * _NS
    rows_per_w = nrows // nw
    ch = 416
    nchunk = rows_per_w // ch
    assert rows_per_w % ch == 0 and nchunk % 2 == 0
    mesh = plsc.VectorSubcoreMesh(core_axis_name="c", subcore_axis_name="s")

    @functools.partial(
        pl.kernel,
        mesh=mesh,
        out_type=jax.ShapeDtypeStruct((nrows, _D), jnp.float32),
        scratch_types=[
            pltpu.VMEM((2, ch), jnp.int32),
            pltpu.VMEM((2, ch), jnp.int32),
            pltpu.VMEM((2, ch, 128), jnp.float32),
            pltpu.VMEM((2, ch, _D), jnp.float32),
            pltpu.SemaphoreType.DMA,
            pltpu.SemaphoreType.DMA,
            pltpu.SemaphoreType.DMA,
            pltpu.SemaphoreType.DMA,
            pltpu.SemaphoreType.DMA,
            pltpu.SemaphoreType.DMA,
        ],
        compiler_params=pltpu.CompilerParams(
            use_tc_tiling_on_sc=False, needs_layout_passes=False),
    )
    def gather_k(idx_hbm, table_hbm, out_hbm, idx_v, grp_v, rows_v, pk_v,
                 is0, is1, gs0, gs1, os0, os1):
        wid = lax.axis_index("s") * _NC + lax.axis_index("c")
        lanes = lax.iota(jnp.int32, 16)
        isem = (is0, is1)
        gsem = (gs0, gs1)
        osem = (os0, os1)

        def start_idx(slot, c):
            base = wid * rows_per_w + c * ch
            pltpu.async_copy(idx_hbm.at[pl.ds(base, ch)], idx_v.at[slot],
                             isem[slot])

        def wait_idx(slot):
            pltpu.make_async_copy(idx_hbm.at[pl.ds(0, ch)], idx_v.at[slot],
                                  isem[slot]).wait()

        def prep(slot):
            wait_idx(slot)

            def to_groups(i, carry):
                v = idx_v[slot, pl.ds(i * 16, 16)]
                grp_v[slot, pl.ds(i * 16, 16)] = v >> 3
                return carry

            lax.fori_loop(0, ch // 16, to_groups, 0, unroll=4)
            pltpu.async_copy(table_hbm.at[grp_v.at[slot]], rows_v.at[slot],
                             gsem[slot])

        def fin(slot, c):
            pltpu.make_async_copy(table_hbm.at[grp_v.at[slot]],
                                  rows_v.at[slot], gsem[slot]).wait()

            @pl.when(c >= 2)
            def _():
                pltpu.make_async_copy(pk_v.at[slot],
                                      out_hbm.at[pl.ds(0, ch)],
                                      osem[slot]).wait()

            def extract(i, carry):
                i0 = i * 16
                rows16 = i0 + lanes
                off = (idx_v[slot, pl.ds(i0, 16)] & 7) * _D
                for k in range(_D):
                    vk = plsc.load_gather(rows_v.at[slot], [rows16, off + k])
                    plsc.store_scatter(pk_v.at[slot], [rows16, lanes * 0 + k],
                                       vk)
                return carry

            lax.fori_loop(0, ch // 16, extract, 0, unroll=2)
            base = wid * rows_per_w + c * ch
            pltpu.async_copy(pk_v.at[slot], out_hbm.at[pl.ds(base, ch)],
                             osem[slot])

            @pl.when(c + 2 < nchunk)
            def _():
                start_idx(slot, c + 2)

        start_idx(0, 0)
        prep(0)
        start_idx(1, 1)

        def pair_loop(p, carry):
            c0 = 2 * p

            @pl.when(c0 + 1 < nchunk)
            def _():
                prep(1)
            fin(0, c0)

            @pl.when(c0 + 2 < nchunk)
            def _():
                prep(0)

            @pl.when(c0 + 1 < nchunk)
            def _():
                fin(1, c0 + 1)
            return carry

        lax.fori_loop(0, nchunk // 2, pair_loop, 0)
        pltpu.make_async_copy(pk_v.at[0], out_hbm.at[pl.ds(0, ch)],
                              osem[0]).wait()
        pltpu.make_async_copy(pk_v.at[1], out_hbm.at[pl.ds(0, ch)],
                              osem[1]).wait()

    return gather_k


def _mlp_body(emb_ref, bias_ref, w1z_ref, u_ref, b1_ref, w2_ref, b2_ref,
              w3t_ref, b3_ref, out_ref):
    e = emb_ref[...]
    zb = (e + bias_ref[...]).astype(jnp.bfloat16)
    acc = jnp.dot(zb, w1z_ref[...], preferred_element_type=jnp.float32)
    eb = e.astype(jnp.bfloat16)
    r = 0
    for d in range(1, _NF):
        w = (_NF - d) * _D
        prod = eb[:, :w] * eb[:, d * _D:]
        acc = acc + jnp.dot(prod, u_ref[r:r + w, :],
                            preferred_element_type=jnp.float32)
        r += w
    h1 = jnp.maximum(acc + b1_ref[...], 0.0).astype(jnp.bfloat16)
    h2 = jnp.maximum(
        jnp.dot(h1, w2_ref[...], preferred_element_type=jnp.float32)
        + b2_ref[...], 0.0)
    logit = jnp.sum(h2 * w3t_ref[...], axis=1, keepdims=True) + b3_ref[...]
    out_ref[...] = jax.nn.sigmoid(logit)


def _mlp(emb2, bias2, w1z, u, b1r, w2b, b2r, w3t, b3r):
    nb = emb2.shape[0]
    tb = 512
    return pl.pallas_call(
        _mlp_body,
        grid=(nb // tb,),
        in_specs=[
            pl.BlockSpec((tb, _ZDIM), lambda i: (i, 0)),
            pl.BlockSpec((1, _ZDIM), lambda i: (0, 0)),
            pl.BlockSpec((_ZDIM, 128), lambda i: (0, 0)),
            pl.BlockSpec((_PDIM, 128), lambda i: (0, 0)),
            pl.BlockSpec((1, 128), lambda i: (0, 0)),
            pl.BlockSpec((128, 64), lambda i: (0, 0)),
            pl.BlockSpec((1, 64), lambda i: (0, 0)),
            pl.BlockSpec((1, 64), lambda i: (0, 0)),
            pl.BlockSpec((1, 1), lambda i: (0, 0)),
        ],
        out_specs=pl.BlockSpec((tb, 1), lambda i: (i, 0)),
        out_shape=jax.ShapeDtypeStruct((nb, 1), jnp.float32),
    )(emb2, bias2, w1z, u, b1r, w2b, b2r, w3t, b3r)


def kernel(x, W_emb, bias, W1, b1, W2, b2, W3, b3):
    offs = jnp.arange(_NF, dtype=jnp.int32) * _FIELD
    idx = (x + offs[None, :]).reshape(-1)

    table_rm = _make_relayout()(W_emb.T)

    w1z = W1[:_ZDIM].astype(jnp.bfloat16)
    u = W1[_ZDIM:][jnp.asarray(_GIDX)].astype(jnp.bfloat16)
    bias2 = bias.reshape(1, _ZDIM)
    b1r = b1.reshape(1, -1)
    b2r = b2.reshape(1, -1)
    w3t = W3.reshape(1, -1)
    b3r = b3.reshape(1, 1)
    w2b = W2.astype(jnp.bfloat16)

    nsplit = 4
    part = _ROWS // nsplit
    gath = _make_gather(part)
    outs = []
    for h in range(nsplit):
        emb = gath(lax.dynamic_slice_in_dim(idx, h * part, part), table_rm)
        emb2 = emb.reshape(_B // nsplit, _ZDIM)
        outs.append(_mlp(emb2, bias2, w1z, u, b1r, w2b, b2r, w3t, b3r))
    return jnp.concatenate(outs, axis=0)
